# Initial kernel scaffold; baseline (speedup 1.0000x reference)
import jax
import jax.numpy as jnp
from jax.experimental import pallas as pl


def kernel(x_base, x_joint, ei_gt, ei_gs, ei_gr, ei_jb, ei_bj, ei_jj, enc_W_base, enc_b_base, enc_W_joint, enc_b_joint, conv_W_rel, conv_b_rel, conv_W_root, bt_W1, bt_b1, bt_W2, bt_b2, dec_W, dec_b):
    raise NotImplementedError("write your pallas kernel here")


# speedup vs baseline: 3.2993x; 3.2993x over previous
import functools

import jax
import jax.numpy as jnp
from jax import lax
from jax.experimental import pallas as pl
from jax.experimental.pallas import tpu as pltpu
from jax.experimental.pallas import tpu_sc as plsc

NC = 2
NS = 16
NW = NC * NS
H = 64
E = 150000
E_PAD = 150016
EPW = E_PAD // NS
NVREG = EPW // 16
G = 128
NBLK = (EPW + G - 1) // G
CAP = NBLK * G
CR = 12800
STRIPE = CR // NS
ZR = 160
NB_PAD = 51200
NJ_PAD = 153600
BR = 2048


def _mesh():
    return plsc.VectorSubcoreMesh(core_axis_name="c", subcore_axis_name="s",
                                  num_cores=NC, num_subcores=NS)


@functools.lru_cache(None)
def _seg_sum(n_src, n_dst_pad):
    cpc = (n_dst_pad // CR) // NC

    @functools.partial(
        pl.kernel,
        out_type=jax.ShapeDtypeStruct((n_dst_pad, H), jnp.float32),
        mesh=_mesh(),
        scratch_types=[
            pltpu.VMEM((EPW,), jnp.int32),
            pltpu.VMEM((EPW,), jnp.int32),
            pltpu.VMEM((CAP,), jnp.int32),
            pltpu.VMEM((CAP,), jnp.int32),
            pltpu.VMEM((NBLK, G), jnp.int32),
            pltpu.VMEM((G, H), jnp.float32),
            pltpu.VMEM((ZR, H), jnp.float32),
            pltpu.VMEM_SHARED((CR + NS, H), jnp.float32),
            pltpu.SemaphoreType.DMA,
        ],
        compiler_params=pltpu.CompilerParams(needs_layout_passes=False, use_tc_tiling_on_sc=False),
    )
    def k(table, src_h, dst_h, out, src_v, dst_v, csrc, cdst, cdst2, rows,
          zbuf, acc, sem):
        c = lax.axis_index("c")
        s = lax.axis_index("s")
        pltpu.sync_copy(src_h.at[pl.ds(s * EPW, EPW)], src_v)
        pltpu.sync_copy(dst_h.at[pl.ds(s * EPW, EPW)], dst_v)

        zf = jnp.zeros((16,), jnp.float32)
        zi = jnp.zeros((16,), jnp.int32)

        def zb_body(i, _):
            for q in range(H // 16):
                zbuf[i, pl.ds(q * 16, 16)] = zf
            return 0
        lax.fori_loop(0, ZR, zb_body, 0)

        dump = CR + s
        dsplat = zi + dump
        for j in range(cpc):
            lo = (c + NC * j) * CR

            def clr(i, _):
                pltpu.sync_copy(zbuf, acc.at[pl.ds(s * STRIPE + i * ZR, ZR)])
                return 0
            lax.fori_loop(0, STRIPE // ZR, clr, 0)

            @pl.when(s == 0)
            def _():
                pltpu.sync_copy(zbuf.at[pl.ds(0, NS)], acc.at[pl.ds(CR, NS)])
            plsc.subcore_barrier()

            def ini(i, _):
                csrc[pl.ds(i * 16, 16)] = zi
                cdst[pl.ds(i * 16, 16)] = dsplat
                return 0
            lax.fori_loop(0, CAP // 16, ini, 0)

            def cmp_body(i, pos):
                d = dst_v[pl.ds(i * 16, 16)]
                si = src_v[pl.ds(i * 16, 16)]
                m = (d >= lo) & (d < lo + CR)
                inc = jnp.where(m, jnp.int32(1), jnp.int32(0))
                at = pos + plsc.cumsum(inc) - 1
                plsc.store_scatter(csrc, [at], si, mask=m)
                plsc.store_scatter(cdst, [at], d - lo, mask=m)
                return pos + jnp.sum(inc)
            pos = lax.fori_loop(0, NVREG, cmp_body, jnp.int32(0))
            nblk = (pos + (G - 1)) // G

            def cp2(b, _):
                for q in range(G // 16):
                    cdst2[b, pl.ds(q * 16, 16)] = cdst[pl.ds(b * G + q * 16, 16)]
                return 0
            lax.fori_loop(0, nblk, cp2, 0)

            def gs(b, _):
                pltpu.async_copy(table.at[csrc.at[pl.ds(b * G, G)]], rows,
                                 sem).wait()
                pltpu.sync_copy(rows, acc.at[cdst2.at[b]], add=True)
                return 0
            lax.fori_loop(0, nblk, gs, 0)
            plsc.subcore_barrier()
            pltpu.sync_copy(acc.at[pl.ds(s * STRIPE, STRIPE)],
                            out.at[pl.ds(lo + s * STRIPE, STRIPE)])
            plsc.subcore_barrier()
    return k


@functools.lru_cache(None)
def _seg_count(n_dst_pad):
    cpc = (n_dst_pad // CR) // NC
    W = 16

    @functools.partial(
        pl.kernel,
        out_type=jax.ShapeDtypeStruct((n_dst_pad, W), jnp.float32),
        mesh=_mesh(),
        scratch_types=[
            pltpu.VMEM((EPW,), jnp.int32),
            pltpu.VMEM((CAP,), jnp.int32),
            pltpu.VMEM((NBLK, G), jnp.int32),
            pltpu.VMEM((G, W), jnp.float32),
            pltpu.VMEM((ZR, W), jnp.float32),
            pltpu.VMEM_SHARED((CR + NS, W), jnp.float32),
        ],
        compiler_params=pltpu.CompilerParams(needs_layout_passes=False, use_tc_tiling_on_sc=False),
    )
    def k(dst_h, out, dst_v, cdst, cdst2, ones, zbuf, acc):
        c = lax.axis_index("c")
        s = lax.axis_index("s")
        pltpu.sync_copy(dst_h.at[pl.ds(s * EPW, EPW)], dst_v)

        zf = jnp.zeros((16,), jnp.float32)
        onef = zf + 1.0
        zi = jnp.zeros((16,), jnp.int32)

        def zb_body(i, _):
            zbuf[i, pl.ds(0, W)] = zf
            return 0
        lax.fori_loop(0, ZR, zb_body, 0)

        def on_body(i, _):
            ones[i, pl.ds(0, W)] = onef
            return 0
        lax.fori_loop(0, G, on_body, 0)

        dump = CR + s
        dsplat = zi + dump
        for j in range(cpc):
            lo = (c + NC * j) * CR

            def clr(i, _):
                pltpu.sync_copy(zbuf, acc.at[pl.ds(s * STRIPE + i * ZR, ZR)])
                return 0
            lax.fori_loop(0, STRIPE // ZR, clr, 0)

            @pl.when(s == 0)
            def _():
                pltpu.sync_copy(zbuf.at[pl.ds(0, NS)], acc.at[pl.ds(CR, NS)])
            plsc.subcore_barrier()

            def ini(i, _):
                cdst[pl.ds(i * 16, 16)] = dsplat
                return 0
            lax.fori_loop(0, CAP // 16, ini, 0)

            def cmp_body(i, pos):
                d = dst_v[pl.ds(i * 16, 16)]
                m = (d >= lo) & (d < lo + CR)
                inc = jnp.where(m, jnp.int32(1), jnp.int32(0))
                at = pos + plsc.cumsum(inc) - 1
                plsc.store_scatter(cdst, [at], d - lo, mask=m)
                return pos + jnp.sum(inc)
            pos = lax.fori_loop(0, NVREG, cmp_body, jnp.int32(0))
            nblk = (pos + (G - 1)) // G

            def cp2(b, _):
                for q in range(G // 16):
                    cdst2[b, pl.ds(q * 16, 16)] = cdst[pl.ds(b * G + q * 16, 16)]
                return 0
            lax.fori_loop(0, nblk, cp2, 0)

            def gs(b, _):
                pltpu.sync_copy(ones, acc.at[cdst2.at[b]], add=True)
                return 0
            lax.fori_loop(0, nblk, gs, 0)
            plsc.subcore_barrier()
            pltpu.sync_copy(acc.at[pl.ds(s * STRIPE, STRIPE)],
                            out.at[pl.ds(lo + s * STRIPE, STRIPE)])
            plsc.subcore_barrier()
    return k


def _enc(x, w, b):
    n, f = x.shape

    def body(x_ref, w_ref, b_ref, o_ref):
        o_ref[...] = jnp.maximum(
            jnp.dot(x_ref[...], w_ref[...], preferred_element_type=jnp.float32)
            + b_ref[...], 0.0)

    return pl.pallas_call(
        body,
        grid=(n // BR,),
        in_specs=[pl.BlockSpec((BR, f), lambda i: (i, 0)),
                  pl.BlockSpec((f, H), lambda i: (0, 0)),
                  pl.BlockSpec((1, H), lambda i: (0, 0))],
        out_specs=pl.BlockSpec((BR, H), lambda i: (i, 0)),
        out_shape=jax.ShapeDtypeStruct((n, H), jnp.float32),
    )(x, w, b)


def _combine_base(a0, a1, a2, a3, c0, c1, xb, wc, bs, wr, w1, b1, w2, b2):
    n = xb.shape[0]

    def body(a0r, a1r, a2r, a3r, c0r, c1r, xr, wcr, bsr, wrr, w1r, b1r, w2r,
             b2r, o_ref):
        r0 = jnp.maximum(c0r[...][:, :1], 1.0)
        r1 = jnp.maximum(c1r[...][:, :1], 1.0)
        acat = jnp.concatenate(
            [a0r[...] / r0, a1r[...] / r1, a2r[...], a3r[...]], axis=1)
        ob = (jnp.dot(acat, wcr[...], preferred_element_type=jnp.float32)
              + bsr[...]
              + jnp.dot(xr[...], wrr[...], preferred_element_type=jnp.float32))
        h = jnp.maximum(
            jnp.dot(ob, w1r[...], preferred_element_type=jnp.float32)
            + b1r[...], 0.0)
        o_ref[...] = (jnp.dot(h, w2r[...], preferred_element_type=jnp.float32)
                      + b2r[...] + xr[...])

    blk = lambda r, cdim: pl.BlockSpec((r, cdim), lambda i: (i, 0))
    full = lambda r, cdim: pl.BlockSpec((r, cdim), lambda i: (0, 0))
    return pl.pallas_call(
        body,
        grid=(n // BR,),
        in_specs=[blk(BR, H), blk(BR, H), blk(BR, H), blk(BR, H),
                  blk(BR, 16), blk(BR, 16), blk(BR, H),
                  full(4 * H, H), full(1, H), full(H, H),
                  full(H, H), full(1, H), full(H, H), full(1, H)],
        out_specs=blk(BR, H),
        out_shape=jax.ShapeDtypeStruct((n, H), jnp.float32),
    )(a0, a1, a2, a3, c0, c1, xb, wc, bs, wr, w1, b1, w2, b2)


def _combine_joint(a0, a1, xj, wc, bs, wr):
    n = xj.shape[0]

    def body(a0r, a1r, xr, wcr, bsr, wrr, o_ref):
        acat = jnp.concatenate([a0r[...], a1r[...]], axis=1)
        oj = (jnp.dot(acat, wcr[...], preferred_element_type=jnp.float32)
              + bsr[...]
              + jnp.dot(xr[...], wrr[...], preferred_element_type=jnp.float32))
        o_ref[...] = jnp.maximum(oj, 0.0) + xr[...]

    blk = lambda r, cdim: pl.BlockSpec((r, cdim), lambda i: (i, 0))
    full = lambda r, cdim: pl.BlockSpec((r, cdim), lambda i: (0, 0))
    return pl.pallas_call(
        body,
        grid=(n // BR,),
        in_specs=[blk(BR, H), blk(BR, H), blk(BR, H),
                  full(2 * H, H), full(1, H), full(H, H)],
        out_specs=blk(BR, H),
        out_shape=jax.ShapeDtypeStruct((n, H), jnp.float32),
    )(a0, a1, xj, wc, bs, wr)


def _decode(xb, w, b):
    n = xb.shape[0]

    def body(x_ref, w_ref, b_ref, o_ref):
        o_ref[...] = (jnp.dot(x_ref[...], w_ref[...],
                              preferred_element_type=jnp.float32) + b_ref[...])

    return pl.pallas_call(
        body,
        grid=(n // BR,),
        in_specs=[pl.BlockSpec((BR, H), lambda i: (i, 0)),
                  pl.BlockSpec((H, 8), lambda i: (0, 0)),
                  pl.BlockSpec((1, 8), lambda i: (0, 0))],
        out_specs=pl.BlockSpec((BR, 8), lambda i: (i, 0)),
        out_shape=jax.ShapeDtypeStruct((n, 8), jnp.float32),
    )(xb, w, b)


def kernel(x_base, x_joint, ei_gt, ei_gs, ei_gr, ei_jb, ei_bj, ei_jj,
           enc_W_base, enc_b_base, enc_W_joint, enc_b_joint,
           conv_W_rel, conv_b_rel, conv_W_root,
           bt_W1, bt_b1, bt_W2, bt_b2, dec_W, dec_b):
    nb = x_base.shape[0]
    nj = x_joint.shape[0]

    xb = _enc(jnp.pad(x_base, ((0, NB_PAD - nb), (0, 8 - x_base.shape[1]))),
              jnp.pad(enc_W_base, ((0, 8 - enc_W_base.shape[0]), (0, 0))),
              enc_b_base[None])
    xj = _enc(jnp.pad(x_joint, ((0, NJ_PAD - nj), (0, 8 - x_joint.shape[1]))),
              jnp.pad(enc_W_joint, ((0, 8 - enc_W_joint.shape[0]), (0, 0))),
              enc_b_joint[None])

    def prep(ei, sentinel):
        src = jnp.pad(ei[0], (0, E_PAD - E))
        dst = jnp.pad(ei[1], (0, E_PAD - E), constant_values=sentinel)
        return src, dst

    sgt = prep(ei_gt, NB_PAD)
    sgs = prep(ei_gs, NB_PAD)
    sgr = prep(ei_gr, NB_PAD)
    sjb = prep(ei_jb, NB_PAD)
    sbj = prep(ei_bj, NJ_PAD)
    sjj = prep(ei_jj, NJ_PAD)

    cgt = _seg_count(NB_PAD)(sgt[1])
    cgs = _seg_count(NB_PAD)(sgs[1])

    for l in range(2):
        a_gt = _seg_sum(NB_PAD, NB_PAD)(xb, *sgt)
        a_gs = _seg_sum(NB_PAD, NB_PAD)(xb, *sgs)
        a_gr = _seg_sum(NB_PAD, NB_PAD)(xb, *sgr)
        a_jb = _seg_sum(NJ_PAD, NB_PAD)(xj, *sjb)
        a_bj = _seg_sum(NB_PAD, NJ_PAD)(xb, *sbj)
        a_jj = _seg_sum(NJ_PAD, NJ_PAD)(xj, *sjj)
        wcb = jnp.concatenate([conv_W_rel[l, 0], conv_W_rel[l, 1],
                               conv_W_rel[l, 2], conv_W_rel[l, 3]], 0)
        wcj = jnp.concatenate([conv_W_rel[l, 4], conv_W_rel[l, 5]], 0)
        bsb = (conv_b_rel[l, 0] + conv_b_rel[l, 1] + conv_b_rel[l, 2]
               + conv_b_rel[l, 3])[None]
        bsj = (conv_b_rel[l, 4] + conv_b_rel[l, 5])[None]
        wrb = (conv_W_root[l, 0] + conv_W_root[l, 1] + conv_W_root[l, 2]
               + conv_W_root[l, 3])
        wrj = conv_W_root[l, 4] + conv_W_root[l, 5]
        xb = _combine_base(a_gt, a_gs, a_gr, a_jb, cgt, cgs, xb, wcb, bsb,
                           wrb, bt_W1, bt_b1[None], bt_W2, bt_b2[None])
        xj = _combine_joint(a_bj, a_jj, xj, wcj, bsj, wrj)

    o = _decode(xb, jnp.pad(dec_W, ((0, 0), (0, 2))),
                jnp.pad(dec_b, (0, 2))[None])
    return o[:nb, :6].reshape(-1, 4, 6)


# speedup vs baseline: 3.5866x; 1.0871x over previous
import functools

import jax
import jax.numpy as jnp
from jax import lax
from jax.experimental import pallas as pl
from jax.experimental.pallas import tpu as pltpu
from jax.experimental.pallas import tpu_sc as plsc

NC = 2
NS = 16
NW = NC * NS
H = 64
E = 150000
E_PAD = 150016
EPW = E_PAD // NS
NVREG = EPW // 16
G = 128
NBLK = (EPW + G - 1) // G
CAP = NBLK * G
CR = 12800
STRIPE = CR // NS
ZR = 160
NB_PAD = 51200
NJ_PAD = 153600
BR = 2048


def _mesh():
    return plsc.VectorSubcoreMesh(core_axis_name="c", subcore_axis_name="s",
                                  num_cores=NC, num_subcores=NS)


@functools.lru_cache(None)
def _seg_sum(n_src, n_dst_pad):
    cpc = (n_dst_pad // CR) // NC

    @functools.partial(
        pl.kernel,
        out_type=jax.ShapeDtypeStruct((n_dst_pad, H), jnp.float32),
        mesh=_mesh(),
        scratch_types=[
            pltpu.VMEM((EPW,), jnp.int32),
            pltpu.VMEM((EPW,), jnp.int32),
            pltpu.VMEM((CAP,), jnp.int32),
            pltpu.VMEM((NBLK, G), jnp.int32),
            pltpu.VMEM((3, G, H), jnp.float32),
            pltpu.VMEM((ZR, H), jnp.float32),
            pltpu.VMEM_SHARED((CR + NS, H), jnp.float32),
            pltpu.SemaphoreType.DMA((3,)),
            pltpu.SemaphoreType.DMA((3,)),
        ],
        compiler_params=pltpu.CompilerParams(needs_layout_passes=False, use_tc_tiling_on_sc=False),
    )
    def k(table, src_h, dst_h, out, src_v, dst_v, csrc, cdst2, rows,
          zbuf, acc, gsem, ssem):
        c = lax.axis_index("c")
        s = lax.axis_index("s")
        pltpu.sync_copy(src_h.at[pl.ds(s * EPW, EPW)], src_v)
        pltpu.sync_copy(dst_h.at[pl.ds(s * EPW, EPW)], dst_v)

        zf = jnp.zeros((16,), jnp.float32)
        zi = jnp.zeros((16,), jnp.int32)
        iota = lax.iota(jnp.int32, 16)

        def zb_body(i, _):
            for q in range(H // 16):
                zbuf[i, pl.ds(q * 16, 16)] = zf
            return 0
        lax.fori_loop(0, ZR, zb_body, 0)

        dump = CR + s
        dsplat = zi + dump
        for j in range(cpc):
            lo = (c + NC * j) * CR

            def clr(i, _):
                pltpu.sync_copy(zbuf, acc.at[pl.ds(s * STRIPE + i * ZR, ZR)])
                return 0
            lax.fori_loop(0, STRIPE // ZR, clr, 0)

            @pl.when(s == 0)
            def _():
                pltpu.sync_copy(zbuf.at[pl.ds(0, NS)], acc.at[pl.ds(CR, NS)])
            plsc.subcore_barrier()

            def cmp_body(i, pos):
                d = dst_v[pl.ds(i * 16, 16)]
                si = src_v[pl.ds(i * 16, 16)]
                m = (d >= lo) & (d < lo + CR)
                inc = jnp.where(m, jnp.int32(1), jnp.int32(0))
                at = pos + plsc.cumsum(inc) - 1
                plsc.store_scatter(csrc, [at], si, mask=m)
                plsc.store_scatter(cdst2, [at >> 7, at & (G - 1)], d - lo,
                                   mask=m)
                return pos + jnp.sum(inc)
            pos = lax.fori_loop(0, NVREG, cmp_body, jnp.int32(0))
            nblk = (pos + (G - 1)) // G

            def tail(kk, _):
                at = pos + kk * 16 + iota
                mm = at < nblk * G
                plsc.store_scatter(csrc, [at], zi, mask=mm)
                plsc.store_scatter(cdst2, [at >> 7, at & (G - 1)], dsplat,
                                   mask=mm)
                return 0
            lax.fori_loop(0, G // 16, tail, 0)

            def g_start(b):
                return pltpu.async_copy(
                    table.at[csrc.at[pl.ds(b * G, G)]], rows.at[b % 3],
                    gsem.at[b % 3])

            @pl.when(nblk > 0)
            def _():
                g_start(0)

            @pl.when(nblk > 1)
            def _():
                g_start(1)

            def gs(b, _):
                pltpu.make_async_copy(
                    table.at[csrc.at[pl.ds(b * G, G)]], rows.at[b % 3],
                    gsem.at[b % 3]).wait()
                pltpu.async_copy(rows.at[b % 3], acc.at[cdst2.at[b]],
                                 ssem.at[b % 3], add=True)

                @pl.when(b >= 1)
                def _():
                    pltpu.make_async_copy(
                        rows.at[(b - 1) % 3], acc.at[cdst2.at[b - 1]],
                        ssem.at[(b - 1) % 3]).wait()

                @pl.when(b + 2 < nblk)
                def _():
                    g_start(b + 2)
                return 0
            lax.fori_loop(0, nblk, gs, 0)

            @pl.when(nblk > 0)
            def _():
                pltpu.make_async_copy(
                    rows.at[(nblk - 1) % 3], acc.at[cdst2.at[nblk - 1]],
                    ssem.at[(nblk - 1) % 3]).wait()
            plsc.subcore_barrier()
            pltpu.sync_copy(acc.at[pl.ds(s * STRIPE, STRIPE)],
                            out.at[pl.ds(lo + s * STRIPE, STRIPE)])
            plsc.subcore_barrier()
    return k


@functools.lru_cache(None)
def _seg_count(n_dst_pad):
    cpc = (n_dst_pad // CR) // NC
    W = 16

    @functools.partial(
        pl.kernel,
        out_type=jax.ShapeDtypeStruct((n_dst_pad, W), jnp.float32),
        mesh=_mesh(),
        scratch_types=[
            pltpu.VMEM((EPW,), jnp.int32),
            pltpu.VMEM((NBLK, G), jnp.int32),
            pltpu.VMEM((G, W), jnp.float32),
            pltpu.VMEM((ZR, W), jnp.float32),
            pltpu.VMEM_SHARED((CR + NS, W), jnp.float32),
        ],
        compiler_params=pltpu.CompilerParams(needs_layout_passes=False, use_tc_tiling_on_sc=False),
    )
    def k(dst_h, out, dst_v, cdst2, ones, zbuf, acc):
        c = lax.axis_index("c")
        s = lax.axis_index("s")
        pltpu.sync_copy(dst_h.at[pl.ds(s * EPW, EPW)], dst_v)

        zf = jnp.zeros((16,), jnp.float32)
        onef = zf + 1.0
        zi = jnp.zeros((16,), jnp.int32)
        iota = lax.iota(jnp.int32, 16)

        def zb_body(i, _):
            zbuf[i, pl.ds(0, W)] = zf
            return 0
        lax.fori_loop(0, ZR, zb_body, 0)

        def on_body(i, _):
            ones[i, pl.ds(0, W)] = onef
            return 0
        lax.fori_loop(0, G, on_body, 0)

        dump = CR + s
        dsplat = zi + dump
        for j in range(cpc):
            lo = (c + NC * j) * CR

            def clr(i, _):
                pltpu.sync_copy(zbuf, acc.at[pl.ds(s * STRIPE + i * ZR, ZR)])
                return 0
            lax.fori_loop(0, STRIPE // ZR, clr, 0)

            @pl.when(s == 0)
            def _():
                pltpu.sync_copy(zbuf.at[pl.ds(0, NS)], acc.at[pl.ds(CR, NS)])
            plsc.subcore_barrier()

            def cmp_body(i, pos):
                d = dst_v[pl.ds(i * 16, 16)]
                m = (d >= lo) & (d < lo + CR)
                inc = jnp.where(m, jnp.int32(1), jnp.int32(0))
                at = pos + plsc.cumsum(inc) - 1
                plsc.store_scatter(cdst2, [at >> 7, at & (G - 1)], d - lo,
                                   mask=m)
                return pos + jnp.sum(inc)
            pos = lax.fori_loop(0, NVREG, cmp_body, jnp.int32(0))
            nblk = (pos + (G - 1)) // G

            def tail(kk, _):
                at = pos + kk * 16 + iota
                mm = at < nblk * G
                plsc.store_scatter(cdst2, [at >> 7, at & (G - 1)], dsplat,
                                   mask=mm)
                return 0
            lax.fori_loop(0, G // 16, tail, 0)

            def gs(b, _):
                pltpu.sync_copy(ones, acc.at[cdst2.at[b]], add=True)
                return 0
            lax.fori_loop(0, nblk, gs, 0)
            plsc.subcore_barrier()
            pltpu.sync_copy(acc.at[pl.ds(s * STRIPE, STRIPE)],
                            out.at[pl.ds(lo + s * STRIPE, STRIPE)])
            plsc.subcore_barrier()
    return k


def _enc(x, w, b):
    n, f = x.shape

    def body(x_ref, w_ref, b_ref, o_ref):
        o_ref[...] = jnp.maximum(
            jnp.dot(x_ref[...], w_ref[...], preferred_element_type=jnp.float32)
            + b_ref[...], 0.0)

    return pl.pallas_call(
        body,
        grid=(n // BR,),
        in_specs=[pl.BlockSpec((BR, f), lambda i: (i, 0)),
                  pl.BlockSpec((f, H), lambda i: (0, 0)),
                  pl.BlockSpec((1, H), lambda i: (0, 0))],
        out_specs=pl.BlockSpec((BR, H), lambda i: (i, 0)),
        out_shape=jax.ShapeDtypeStruct((n, H), jnp.float32),
    )(x, w, b)


def _combine_base(a0, a1, a2, a3, c0, c1, xb, wc, bs, wr, w1, b1, w2, b2):
    n = xb.shape[0]

    def body(a0r, a1r, a2r, a3r, c0r, c1r, xr, wcr, bsr, wrr, w1r, b1r, w2r,
             b2r, o_ref):
        r0 = jnp.maximum(c0r[...][:, :1], 1.0)
        r1 = jnp.maximum(c1r[...][:, :1], 1.0)
        acat = jnp.concatenate(
            [a0r[...] / r0, a1r[...] / r1, a2r[...], a3r[...]], axis=1)
        ob = (jnp.dot(acat, wcr[...], preferred_element_type=jnp.float32)
              + bsr[...]
              + jnp.dot(xr[...], wrr[...], preferred_element_type=jnp.float32))
        h = jnp.maximum(
            jnp.dot(ob, w1r[...], preferred_element_type=jnp.float32)
            + b1r[...], 0.0)
        o_ref[...] = (jnp.dot(h, w2r[...], preferred_element_type=jnp.float32)
                      + b2r[...] + xr[...])

    blk = lambda r, cdim: pl.BlockSpec((r, cdim), lambda i: (i, 0))
    full = lambda r, cdim: pl.BlockSpec((r, cdim), lambda i: (0, 0))
    return pl.pallas_call(
        body,
        grid=(n // BR,),
        in_specs=[blk(BR, H), blk(BR, H), blk(BR, H), blk(BR, H),
                  blk(BR, 16), blk(BR, 16), blk(BR, H),
                  full(4 * H, H), full(1, H), full(H, H),
                  full(H, H), full(1, H), full(H, H), full(1, H)],
        out_specs=blk(BR, H),
        out_shape=jax.ShapeDtypeStruct((n, H), jnp.float32),
    )(a0, a1, a2, a3, c0, c1, xb, wc, bs, wr, w1, b1, w2, b2)


def _combine_joint(a0, a1, xj, wc, bs, wr):
    n = xj.shape[0]

    def body(a0r, a1r, xr, wcr, bsr, wrr, o_ref):
        acat = jnp.concatenate([a0r[...], a1r[...]], axis=1)
        oj = (jnp.dot(acat, wcr[...], preferred_element_type=jnp.float32)
              + bsr[...]
              + jnp.dot(xr[...], wrr[...], preferred_element_type=jnp.float32))
        o_ref[...] = jnp.maximum(oj, 0.0) + xr[...]

    blk = lambda r, cdim: pl.BlockSpec((r, cdim), lambda i: (i, 0))
    full = lambda r, cdim: pl.BlockSpec((r, cdim), lambda i: (0, 0))
    return pl.pallas_call(
        body,
        grid=(n // BR,),
        in_specs=[blk(BR, H), blk(BR, H), blk(BR, H),
                  full(2 * H, H), full(1, H), full(H, H)],
        out_specs=blk(BR, H),
        out_shape=jax.ShapeDtypeStruct((n, H), jnp.float32),
    )(a0, a1, xj, wc, bs, wr)


def _decode(xb, w, b):
    n = xb.shape[0]

    def body(x_ref, w_ref, b_ref, o_ref):
        o_ref[...] = (jnp.dot(x_ref[...], w_ref[...],
                              preferred_element_type=jnp.float32) + b_ref[...])

    return pl.pallas_call(
        body,
        grid=(n // BR,),
        in_specs=[pl.BlockSpec((BR, H), lambda i: (i, 0)),
                  pl.BlockSpec((H, 8), lambda i: (0, 0)),
                  pl.BlockSpec((1, 8), lambda i: (0, 0))],
        out_specs=pl.BlockSpec((BR, 8), lambda i: (i, 0)),
        out_shape=jax.ShapeDtypeStruct((n, 8), jnp.float32),
    )(xb, w, b)


def kernel(x_base, x_joint, ei_gt, ei_gs, ei_gr, ei_jb, ei_bj, ei_jj,
           enc_W_base, enc_b_base, enc_W_joint, enc_b_joint,
           conv_W_rel, conv_b_rel, conv_W_root,
           bt_W1, bt_b1, bt_W2, bt_b2, dec_W, dec_b):
    nb = x_base.shape[0]
    nj = x_joint.shape[0]

    xb = _enc(jnp.pad(x_base, ((0, NB_PAD - nb), (0, 8 - x_base.shape[1]))),
              jnp.pad(enc_W_base, ((0, 8 - enc_W_base.shape[0]), (0, 0))),
              enc_b_base[None])
    xj = _enc(jnp.pad(x_joint, ((0, NJ_PAD - nj), (0, 8 - x_joint.shape[1]))),
              jnp.pad(enc_W_joint, ((0, 8 - enc_W_joint.shape[0]), (0, 0))),
              enc_b_joint[None])

    def prep(ei, sentinel):
        src = jnp.pad(ei[0], (0, E_PAD - E))
        dst = jnp.pad(ei[1], (0, E_PAD - E), constant_values=sentinel)
        return src, dst

    sgt = prep(ei_gt, NB_PAD)
    sgs = prep(ei_gs, NB_PAD)
    sgr = prep(ei_gr, NB_PAD)
    sjb = prep(ei_jb, NB_PAD)
    sbj = prep(ei_bj, NJ_PAD)
    sjj = prep(ei_jj, NJ_PAD)

    cgt = _seg_count(NB_PAD)(sgt[1])
    cgs = _seg_count(NB_PAD)(sgs[1])

    for l in range(2):
        a_gt = _seg_sum(NB_PAD, NB_PAD)(xb, *sgt)
        a_gs = _seg_sum(NB_PAD, NB_PAD)(xb, *sgs)
        a_gr = _seg_sum(NB_PAD, NB_PAD)(xb, *sgr)
        a_jb = _seg_sum(NJ_PAD, NB_PAD)(xj, *sjb)
        a_bj = _seg_sum(NB_PAD, NJ_PAD)(xb, *sbj)
        a_jj = _seg_sum(NJ_PAD, NJ_PAD)(xj, *sjj)
        wcb = jnp.concatenate([conv_W_rel[l, 0], conv_W_rel[l, 1],
                               conv_W_rel[l, 2], conv_W_rel[l, 3]], 0)
        wcj = jnp.concatenate([conv_W_rel[l, 4], conv_W_rel[l, 5]], 0)
        bsb = (conv_b_rel[l, 0] + conv_b_rel[l, 1] + conv_b_rel[l, 2]
               + conv_b_rel[l, 3])[None]
        bsj = (conv_b_rel[l, 4] + conv_b_rel[l, 5])[None]
        wrb = (conv_W_root[l, 0] + conv_W_root[l, 1] + conv_W_root[l, 2]
               + conv_W_root[l, 3])
        wrj = conv_W_root[l, 4] + conv_W_root[l, 5]
        xb = _combine_base(a_gt, a_gs, a_gr, a_jb, cgt, cgs, xb, wcb, bsb,
                           wrb, bt_W1, bt_b1[None], bt_W2, bt_b2[None])
        xj = _combine_joint(a_bj, a_jj, xj, wcj, bsj, wrj)

    o = _decode(xb, jnp.pad(dec_W, ((0, 0), (0, 2))),
                jnp.pad(dec_b, (0, 2))[None])
    return o[:nb, :6].reshape(-1, 4, 6)


# speedup vs baseline: 3.6090x; 1.0063x over previous
import functools

import jax
import jax.numpy as jnp
from jax import lax
from jax.experimental import pallas as pl
from jax.experimental.pallas import tpu as pltpu
from jax.experimental.pallas import tpu_sc as plsc

NC = 2
NS = 16
NW = NC * NS
H = 64
E = 150000
E_PAD = 150016
EPW = E_PAD // NS
NVREG = EPW // 16
G = 128
NBLK = (EPW + G - 1) // G
CAP = NBLK * G
CR = 12800
STRIPE = CR // NS
ZR = 160
NB_PAD = 51200
NJ_PAD = 153600
BR = 2048


def _mesh():
    return plsc.VectorSubcoreMesh(core_axis_name="c", subcore_axis_name="s",
                                  num_cores=NC, num_subcores=NS)


@functools.lru_cache(None)
def _seg_sum(n_src, n_dst_pad):
    cpc = (n_dst_pad // CR) // NC

    @functools.partial(
        pl.kernel,
        out_type=jax.ShapeDtypeStruct((n_dst_pad, H), jnp.float32),
        mesh=_mesh(),
        scratch_types=[
            pltpu.VMEM((EPW,), jnp.int32),
            pltpu.VMEM((EPW,), jnp.int32),
            pltpu.VMEM((CAP,), jnp.int32),
            pltpu.VMEM((NBLK, G), jnp.int32),
            pltpu.VMEM((3, G, H), jnp.float32),
            pltpu.VMEM((ZR, H), jnp.float32),
            pltpu.VMEM_SHARED((CR + NS, H), jnp.float32),
            pltpu.SemaphoreType.DMA((3,)),
            pltpu.SemaphoreType.DMA((3,)),
        ],
        compiler_params=pltpu.CompilerParams(needs_layout_passes=False, use_tc_tiling_on_sc=False),
    )
    def k(table, src_h, dst_h, out, src_v, dst_v, csrc, cdst2, rows,
          zbuf, acc, gsem, ssem):
        c = lax.axis_index("c")
        s = lax.axis_index("s")
        pltpu.sync_copy(src_h.at[pl.ds(s * EPW, EPW)], src_v)
        pltpu.sync_copy(dst_h.at[pl.ds(s * EPW, EPW)], dst_v)

        zf = jnp.zeros((16,), jnp.float32)
        zi = jnp.zeros((16,), jnp.int32)
        iota = lax.iota(jnp.int32, 16)

        def zb_body(i, _):
            for q in range(H // 16):
                zbuf[i, pl.ds(q * 16, 16)] = zf
            return 0
        lax.fori_loop(0, ZR, zb_body, 0)

        dump = CR + s
        dsplat = zi + dump
        for j in range(cpc):
            lo = (c + NC * j) * CR

            def clr(i, _):
                pltpu.sync_copy(zbuf, acc.at[pl.ds(s * STRIPE + i * ZR, ZR)])
                return 0
            lax.fori_loop(0, STRIPE // ZR, clr, 0)

            @pl.when(s == 0)
            def _():
                pltpu.sync_copy(zbuf.at[pl.ds(0, NS)], acc.at[pl.ds(CR, NS)])
            plsc.subcore_barrier()

            def cmp_body(i, pos):
                d1 = dst_v[pl.ds(i * 32, 16)]
                s1 = src_v[pl.ds(i * 32, 16)]
                d2 = dst_v[pl.ds(i * 32 + 16, 16)]
                s2 = src_v[pl.ds(i * 32 + 16, 16)]
                m1 = (d1 >= lo) & (d1 < lo + CR)
                m2 = (d2 >= lo) & (d2 < lo + CR)
                cs1 = plsc.cumsum(jnp.where(m1, jnp.int32(1), jnp.int32(0)))
                cs2 = plsc.cumsum(jnp.where(m2, jnp.int32(1), jnp.int32(0)))
                c1 = cs1[15]
                at1 = pos + cs1 - 1
                at2 = pos + c1 + cs2 - 1
                plsc.store_scatter(csrc, [at1], s1, mask=m1)
                plsc.store_scatter(cdst2, [at1 >> 7, at1 & (G - 1)], d1 - lo,
                                   mask=m1)
                plsc.store_scatter(csrc, [at2], s2, mask=m2)
                plsc.store_scatter(cdst2, [at2 >> 7, at2 & (G - 1)], d2 - lo,
                                   mask=m2)
                return pos + c1 + cs2[15]
            pos = lax.fori_loop(0, NVREG // 2, cmp_body, jnp.int32(0))
            nblk = (pos + (G - 1)) // G

            def tail(kk, _):
                at = pos + kk * 16 + iota
                mm = at < nblk * G
                plsc.store_scatter(csrc, [at], zi, mask=mm)
                plsc.store_scatter(cdst2, [at >> 7, at & (G - 1)], dsplat,
                                   mask=mm)
                return 0
            lax.fori_loop(0, G // 16, tail, 0)

            def g_start(b):
                return pltpu.async_copy(
                    table.at[csrc.at[pl.ds(b * G, G)]], rows.at[b % 3],
                    gsem.at[b % 3])

            @pl.when(nblk > 0)
            def _():
                g_start(0)

            @pl.when(nblk > 1)
            def _():
                g_start(1)

            def gs(b, _):
                pltpu.make_async_copy(
                    table.at[csrc.at[pl.ds(b * G, G)]], rows.at[b % 3],
                    gsem.at[b % 3]).wait()
                pltpu.async_copy(rows.at[b % 3], acc.at[cdst2.at[b]],
                                 ssem.at[b % 3], add=True)

                @pl.when(b >= 1)
                def _():
                    pltpu.make_async_copy(
                        rows.at[(b - 1) % 3], acc.at[cdst2.at[b - 1]],
                        ssem.at[(b - 1) % 3]).wait()

                @pl.when(b + 2 < nblk)
                def _():
                    g_start(b + 2)
                return 0
            lax.fori_loop(0, nblk, gs, 0)

            @pl.when(nblk > 0)
            def _():
                pltpu.make_async_copy(
                    rows.at[(nblk - 1) % 3], acc.at[cdst2.at[nblk - 1]],
                    ssem.at[(nblk - 1) % 3]).wait()
            plsc.subcore_barrier()
            pltpu.sync_copy(acc.at[pl.ds(s * STRIPE, STRIPE)],
                            out.at[pl.ds(lo + s * STRIPE, STRIPE)])
            plsc.subcore_barrier()
    return k


@functools.lru_cache(None)
def _seg_count(n_dst_pad):
    cpc = (n_dst_pad // CR) // NC
    W = 16

    @functools.partial(
        pl.kernel,
        out_type=jax.ShapeDtypeStruct((n_dst_pad, W), jnp.float32),
        mesh=_mesh(),
        scratch_types=[
            pltpu.VMEM((EPW,), jnp.int32),
            pltpu.VMEM((NBLK, G), jnp.int32),
            pltpu.VMEM((G, W), jnp.float32),
            pltpu.VMEM((ZR, W), jnp.float32),
            pltpu.VMEM_SHARED((CR + NS, W), jnp.float32),
        ],
        compiler_params=pltpu.CompilerParams(needs_layout_passes=False, use_tc_tiling_on_sc=False),
    )
    def k(dst_h, out, dst_v, cdst2, ones, zbuf, acc):
        c = lax.axis_index("c")
        s = lax.axis_index("s")
        pltpu.sync_copy(dst_h.at[pl.ds(s * EPW, EPW)], dst_v)

        zf = jnp.zeros((16,), jnp.float32)
        onef = zf + 1.0
        zi = jnp.zeros((16,), jnp.int32)
        iota = lax.iota(jnp.int32, 16)

        def zb_body(i, _):
            zbuf[i, pl.ds(0, W)] = zf
            return 0
        lax.fori_loop(0, ZR, zb_body, 0)

        def on_body(i, _):
            ones[i, pl.ds(0, W)] = onef
            return 0
        lax.fori_loop(0, G, on_body, 0)

        dump = CR + s
        dsplat = zi + dump
        for j in range(cpc):
            lo = (c + NC * j) * CR

            def clr(i, _):
                pltpu.sync_copy(zbuf, acc.at[pl.ds(s * STRIPE + i * ZR, ZR)])
                return 0
            lax.fori_loop(0, STRIPE // ZR, clr, 0)

            @pl.when(s == 0)
            def _():
                pltpu.sync_copy(zbuf.at[pl.ds(0, NS)], acc.at[pl.ds(CR, NS)])
            plsc.subcore_barrier()

            def cmp_body(i, pos):
                d1 = dst_v[pl.ds(i * 32, 16)]
                d2 = dst_v[pl.ds(i * 32 + 16, 16)]
                m1 = (d1 >= lo) & (d1 < lo + CR)
                m2 = (d2 >= lo) & (d2 < lo + CR)
                cs1 = plsc.cumsum(jnp.where(m1, jnp.int32(1), jnp.int32(0)))
                cs2 = plsc.cumsum(jnp.where(m2, jnp.int32(1), jnp.int32(0)))
                c1 = cs1[15]
                at1 = pos + cs1 - 1
                at2 = pos + c1 + cs2 - 1
                plsc.store_scatter(cdst2, [at1 >> 7, at1 & (G - 1)], d1 - lo,
                                   mask=m1)
                plsc.store_scatter(cdst2, [at2 >> 7, at2 & (G - 1)], d2 - lo,
                                   mask=m2)
                return pos + c1 + cs2[15]
            pos = lax.fori_loop(0, NVREG // 2, cmp_body, jnp.int32(0))
            nblk = (pos + (G - 1)) // G

            def tail(kk, _):
                at = pos + kk * 16 + iota
                mm = at < nblk * G
                plsc.store_scatter(cdst2, [at >> 7, at & (G - 1)], dsplat,
                                   mask=mm)
                return 0
            lax.fori_loop(0, G // 16, tail, 0)

            def gs(b, _):
                pltpu.sync_copy(ones, acc.at[cdst2.at[b]], add=True)
                return 0
            lax.fori_loop(0, nblk, gs, 0)
            plsc.subcore_barrier()
            pltpu.sync_copy(acc.at[pl.ds(s * STRIPE, STRIPE)],
                            out.at[pl.ds(lo + s * STRIPE, STRIPE)])
            plsc.subcore_barrier()
    return k


def _enc(x, w, b):
    n, f = x.shape

    def body(x_ref, w_ref, b_ref, o_ref):
        o_ref[...] = jnp.maximum(
            jnp.dot(x_ref[...], w_ref[...], preferred_element_type=jnp.float32)
            + b_ref[...], 0.0)

    return pl.pallas_call(
        body,
        grid=(n // BR,),
        in_specs=[pl.BlockSpec((BR, f), lambda i: (i, 0)),
                  pl.BlockSpec((f, H), lambda i: (0, 0)),
                  pl.BlockSpec((1, H), lambda i: (0, 0))],
        out_specs=pl.BlockSpec((BR, H), lambda i: (i, 0)),
        out_shape=jax.ShapeDtypeStruct((n, H), jnp.float32),
    )(x, w, b)


def _combine_base(a0, a1, a2, a3, c0, c1, xb, wc, bs, wr, w1, b1, w2, b2):
    n = xb.shape[0]

    def body(a0r, a1r, a2r, a3r, c0r, c1r, xr, wcr, bsr, wrr, w1r, b1r, w2r,
             b2r, o_ref):
        r0 = jnp.maximum(c0r[...][:, :1], 1.0)
        r1 = jnp.maximum(c1r[...][:, :1], 1.0)
        acat = jnp.concatenate(
            [a0r[...] / r0, a1r[...] / r1, a2r[...], a3r[...]], axis=1)
        ob = (jnp.dot(acat, wcr[...], preferred_element_type=jnp.float32)
              + bsr[...]
              + jnp.dot(xr[...], wrr[...], preferred_element_type=jnp.float32))
        h = jnp.maximum(
            jnp.dot(ob, w1r[...], preferred_element_type=jnp.float32)
            + b1r[...], 0.0)
        o_ref[...] = (jnp.dot(h, w2r[...], preferred_element_type=jnp.float32)
                      + b2r[...] + xr[...])

    blk = lambda r, cdim: pl.BlockSpec((r, cdim), lambda i: (i, 0))
    full = lambda r, cdim: pl.BlockSpec((r, cdim), lambda i: (0, 0))
    return pl.pallas_call(
        body,
        grid=(n // BR,),
        in_specs=[blk(BR, H), blk(BR, H), blk(BR, H), blk(BR, H),
                  blk(BR, 16), blk(BR, 16), blk(BR, H),
                  full(4 * H, H), full(1, H), full(H, H),
                  full(H, H), full(1, H), full(H, H), full(1, H)],
        out_specs=blk(BR, H),
        out_shape=jax.ShapeDtypeStruct((n, H), jnp.float32),
    )(a0, a1, a2, a3, c0, c1, xb, wc, bs, wr, w1, b1, w2, b2)


def _combine_joint(a0, a1, xj, wc, bs, wr):
    n = xj.shape[0]

    def body(a0r, a1r, xr, wcr, bsr, wrr, o_ref):
        acat = jnp.concatenate([a0r[...], a1r[...]], axis=1)
        oj = (jnp.dot(acat, wcr[...], preferred_element_type=jnp.float32)
              + bsr[...]
              + jnp.dot(xr[...], wrr[...], preferred_element_type=jnp.float32))
        o_ref[...] = jnp.maximum(oj, 0.0) + xr[...]

    blk = lambda r, cdim: pl.BlockSpec((r, cdim), lambda i: (i, 0))
    full = lambda r, cdim: pl.BlockSpec((r, cdim), lambda i: (0, 0))
    return pl.pallas_call(
        body,
        grid=(n // BR,),
        in_specs=[blk(BR, H), blk(BR, H), blk(BR, H),
                  full(2 * H, H), full(1, H), full(H, H)],
        out_specs=blk(BR, H),
        out_shape=jax.ShapeDtypeStruct((n, H), jnp.float32),
    )(a0, a1, xj, wc, bs, wr)


def _decode(xb, w, b):
    n = xb.shape[0]

    def body(x_ref, w_ref, b_ref, o_ref):
        o_ref[...] = (jnp.dot(x_ref[...], w_ref[...],
                              preferred_element_type=jnp.float32) + b_ref[...])

    return pl.pallas_call(
        body,
        grid=(n // BR,),
        in_specs=[pl.BlockSpec((BR, H), lambda i: (i, 0)),
                  pl.BlockSpec((H, 8), lambda i: (0, 0)),
                  pl.BlockSpec((1, 8), lambda i: (0, 0))],
        out_specs=pl.BlockSpec((BR, 8), lambda i: (i, 0)),
        out_shape=jax.ShapeDtypeStruct((n, 8), jnp.float32),
    )(xb, w, b)


def kernel(x_base, x_joint, ei_gt, ei_gs, ei_gr, ei_jb, ei_bj, ei_jj,
           enc_W_base, enc_b_base, enc_W_joint, enc_b_joint,
           conv_W_rel, conv_b_rel, conv_W_root,
           bt_W1, bt_b1, bt_W2, bt_b2, dec_W, dec_b):
    nb = x_base.shape[0]
    nj = x_joint.shape[0]

    xb = _enc(jnp.pad(x_base, ((0, NB_PAD - nb), (0, 8 - x_base.shape[1]))),
              jnp.pad(enc_W_base, ((0, 8 - enc_W_base.shape[0]), (0, 0))),
              enc_b_base[None])
    xj = _enc(jnp.pad(x_joint, ((0, NJ_PAD - nj), (0, 8 - x_joint.shape[1]))),
              jnp.pad(enc_W_joint, ((0, 8 - enc_W_joint.shape[0]), (0, 0))),
              enc_b_joint[None])

    def prep(ei, sentinel):
        src = jnp.pad(ei[0], (0, E_PAD - E))
        dst = jnp.pad(ei[1], (0, E_PAD - E), constant_values=sentinel)
        return src, dst

    sgt = prep(ei_gt, NB_PAD)
    sgs = prep(ei_gs, NB_PAD)
    sgr = prep(ei_gr, NB_PAD)
    sjb = prep(ei_jb, NB_PAD)
    sbj = prep(ei_bj, NJ_PAD)
    sjj = prep(ei_jj, NJ_PAD)

    cgt = _seg_count(NB_PAD)(sgt[1])
    cgs = _seg_count(NB_PAD)(sgs[1])

    for l in range(2):
        a_gt = _seg_sum(NB_PAD, NB_PAD)(xb, *sgt)
        a_gs = _seg_sum(NB_PAD, NB_PAD)(xb, *sgs)
        a_gr = _seg_sum(NB_PAD, NB_PAD)(xb, *sgr)
        a_jb = _seg_sum(NJ_PAD, NB_PAD)(xj, *sjb)
        a_bj = _seg_sum(NB_PAD, NJ_PAD)(xb, *sbj)
        a_jj = _seg_sum(NJ_PAD, NJ_PAD)(xj, *sjj)
        wcb = jnp.concatenate([conv_W_rel[l, 0], conv_W_rel[l, 1],
                               conv_W_rel[l, 2], conv_W_rel[l, 3]], 0)
        wcj = jnp.concatenate([conv_W_rel[l, 4], conv_W_rel[l, 5]], 0)
        bsb = (conv_b_rel[l, 0] + conv_b_rel[l, 1] + conv_b_rel[l, 2]
               + conv_b_rel[l, 3])[None]
        bsj = (conv_b_rel[l, 4] + conv_b_rel[l, 5])[None]
        wrb = (conv_W_root[l, 0] + conv_W_root[l, 1] + conv_W_root[l, 2]
               + conv_W_root[l, 3])
        wrj = conv_W_root[l, 4] + conv_W_root[l, 5]
        xb = _combine_base(a_gt, a_gs, a_gr, a_jb, cgt, cgs, xb, wcb, bsb,
                           wrb, bt_W1, bt_b1[None], bt_W2, bt_b2[None])
        xj = _combine_joint(a_bj, a_jj, xj, wcj, bsj, wrj)

    o = _decode(xb, jnp.pad(dec_W, ((0, 0), (0, 2))),
                jnp.pad(dec_b, (0, 2))[None])
    return o[:nb, :6].reshape(-1, 4, 6)


# speedup vs baseline: 3.6573x; 1.0134x over previous
import functools

import jax
import jax.numpy as jnp
from jax import lax
from jax.experimental import pallas as pl
from jax.experimental.pallas import tpu as pltpu
from jax.experimental.pallas import tpu_sc as plsc

NC = 2
NS = 16
NW = NC * NS
H = 64
E = 150000
E_PAD = 150016
EPW = E_PAD // NS
NVREG = EPW // 16
G = 128
NBLK = (EPW + G - 1) // G
CAP = NBLK * G
CR = 12800
STRIPE = CR // NS
ZR = 160
NB_PAD = 51200
NJ_PAD = 153600
BR = 2048


def _mesh():
    return plsc.VectorSubcoreMesh(core_axis_name="c", subcore_axis_name="s",
                                  num_cores=NC, num_subcores=NS)


@functools.lru_cache(None)
def _seg_sum(n_src, n_dst_pad):
    cpc = (n_dst_pad // CR) // NC

    @functools.partial(
        pl.kernel,
        out_type=jax.ShapeDtypeStruct((n_dst_pad, H), jnp.float32),
        mesh=_mesh(),
        scratch_types=[
            pltpu.VMEM((EPW,), jnp.int32),
            pltpu.VMEM((EPW,), jnp.int32),
            pltpu.VMEM((CAP,), jnp.int32),
            pltpu.VMEM((NBLK, G), jnp.int32),
            pltpu.VMEM((3, G, H), jnp.float32),
            pltpu.VMEM((ZR, H), jnp.float32),
            pltpu.VMEM_SHARED((CR + NS, H), jnp.float32),
            pltpu.SemaphoreType.DMA((3,)),
            pltpu.SemaphoreType.DMA((3,)),
            pltpu.SemaphoreType.DMA,
        ],
        compiler_params=pltpu.CompilerParams(needs_layout_passes=False, use_tc_tiling_on_sc=False),
    )
    def k(table, src_h, dst_h, out, src_v, dst_v, csrc, cdst2, rows,
          zbuf, acc, gsem, ssem, zsem):
        c = lax.axis_index("c")
        s = lax.axis_index("s")
        pltpu.sync_copy(src_h.at[pl.ds(s * EPW, EPW)], src_v)
        pltpu.sync_copy(dst_h.at[pl.ds(s * EPW, EPW)], dst_v)

        zf = jnp.zeros((16,), jnp.float32)
        zi = jnp.zeros((16,), jnp.int32)
        iota = lax.iota(jnp.int32, 16)

        def zb_body(i, _):
            for q in range(H // 16):
                zbuf[i, pl.ds(q * 16, 16)] = zf
            return 0
        lax.fori_loop(0, ZR, zb_body, 0)

        dump = CR + s
        dsplat = zi + dump
        for j in range(cpc):
            lo = (c + NC * j) * CR

            for i in range(STRIPE // ZR):
                pltpu.async_copy(zbuf, acc.at[pl.ds(s * STRIPE + i * ZR, ZR)],
                                 zsem)

            @pl.when(s == 0)
            def _():
                pltpu.async_copy(zbuf.at[pl.ds(0, NS)], acc.at[pl.ds(CR, NS)],
                                 zsem)

            def cmp_body(i, pos):
                d1 = dst_v[pl.ds(i * 32, 16)]
                s1 = src_v[pl.ds(i * 32, 16)]
                d2 = dst_v[pl.ds(i * 32 + 16, 16)]
                s2 = src_v[pl.ds(i * 32 + 16, 16)]
                m1 = (d1 >= lo) & (d1 < lo + CR)
                m2 = (d2 >= lo) & (d2 < lo + CR)
                cs1 = plsc.cumsum(jnp.where(m1, jnp.int32(1), jnp.int32(0)))
                cs2 = plsc.cumsum(jnp.where(m2, jnp.int32(1), jnp.int32(0)))
                c1 = cs1[15]
                at1 = pos + cs1 - 1
                at2 = pos + c1 + cs2 - 1
                plsc.store_scatter(csrc, [at1], s1, mask=m1)
                plsc.store_scatter(cdst2, [at1 >> 7, at1 & (G - 1)], d1 - lo,
                                   mask=m1)
                plsc.store_scatter(csrc, [at2], s2, mask=m2)
                plsc.store_scatter(cdst2, [at2 >> 7, at2 & (G - 1)], d2 - lo,
                                   mask=m2)
                return pos + c1 + cs2[15]
            pos = lax.fori_loop(0, NVREG // 2, cmp_body, jnp.int32(0))
            nblk = (pos + (G - 1)) // G

            def tail(kk, _):
                at = pos + kk * 16 + iota
                mm = at < nblk * G
                plsc.store_scatter(csrc, [at], zi, mask=mm)
                plsc.store_scatter(cdst2, [at >> 7, at & (G - 1)], dsplat,
                                   mask=mm)
                return 0
            lax.fori_loop(0, G // 16, tail, 0)

            for i in range(STRIPE // ZR):
                pltpu.make_async_copy(
                    zbuf, acc.at[pl.ds(s * STRIPE + i * ZR, ZR)], zsem).wait()

            @pl.when(s == 0)
            def _():
                pltpu.make_async_copy(
                    zbuf.at[pl.ds(0, NS)], acc.at[pl.ds(CR, NS)], zsem).wait()
            plsc.subcore_barrier()

            def g_start(b):
                return pltpu.async_copy(
                    table.at[csrc.at[pl.ds(b * G, G)]], rows.at[b % 3],
                    gsem.at[b % 3])

            @pl.when(nblk > 0)
            def _():
                g_start(0)

            @pl.when(nblk > 1)
            def _():
                g_start(1)

            def gs(b, _):
                pltpu.make_async_copy(
                    table.at[csrc.at[pl.ds(b * G, G)]], rows.at[b % 3],
                    gsem.at[b % 3]).wait()
                pltpu.async_copy(rows.at[b % 3], acc.at[cdst2.at[b]],
                                 ssem.at[b % 3], add=True)

                @pl.when(b >= 1)
                def _():
                    pltpu.make_async_copy(
                        rows.at[(b - 1) % 3], acc.at[cdst2.at[b - 1]],
                        ssem.at[(b - 1) % 3]).wait()

                @pl.when(b + 2 < nblk)
                def _():
                    g_start(b + 2)
                return 0
            lax.fori_loop(0, nblk, gs, 0)

            @pl.when(nblk > 0)
            def _():
                pltpu.make_async_copy(
                    rows.at[(nblk - 1) % 3], acc.at[cdst2.at[nblk - 1]],
                    ssem.at[(nblk - 1) % 3]).wait()
            plsc.subcore_barrier()
            pltpu.sync_copy(acc.at[pl.ds(s * STRIPE, STRIPE)],
                            out.at[pl.ds(lo + s * STRIPE, STRIPE)])
            plsc.subcore_barrier()
    return k


@functools.lru_cache(None)
def _seg_count(n_dst_pad):
    cpc = (n_dst_pad // CR) // NC
    W = 16

    @functools.partial(
        pl.kernel,
        out_type=jax.ShapeDtypeStruct((n_dst_pad, W), jnp.float32),
        mesh=_mesh(),
        scratch_types=[
            pltpu.VMEM((EPW,), jnp.int32),
            pltpu.VMEM((NBLK, G), jnp.int32),
            pltpu.VMEM((G, W), jnp.float32),
            pltpu.VMEM((ZR, W), jnp.float32),
            pltpu.VMEM_SHARED((CR + NS, W), jnp.float32),
            pltpu.SemaphoreType.DMA,
        ],
        compiler_params=pltpu.CompilerParams(needs_layout_passes=False, use_tc_tiling_on_sc=False),
    )
    def k(dst_h, out, dst_v, cdst2, ones, zbuf, acc, zsem):
        c = lax.axis_index("c")
        s = lax.axis_index("s")
        pltpu.sync_copy(dst_h.at[pl.ds(s * EPW, EPW)], dst_v)

        zf = jnp.zeros((16,), jnp.float32)
        onef = zf + 1.0
        zi = jnp.zeros((16,), jnp.int32)
        iota = lax.iota(jnp.int32, 16)

        def zb_body(i, _):
            zbuf[i, pl.ds(0, W)] = zf
            return 0
        lax.fori_loop(0, ZR, zb_body, 0)

        def on_body(i, _):
            ones[i, pl.ds(0, W)] = onef
            return 0
        lax.fori_loop(0, G, on_body, 0)

        dump = CR + s
        dsplat = zi + dump
        for j in range(cpc):
            lo = (c + NC * j) * CR

            for i in range(STRIPE // ZR):
                pltpu.async_copy(zbuf, acc.at[pl.ds(s * STRIPE + i * ZR, ZR)],
                                 zsem)

            @pl.when(s == 0)
            def _():
                pltpu.async_copy(zbuf.at[pl.ds(0, NS)], acc.at[pl.ds(CR, NS)],
                                 zsem)

            def cmp_body(i, pos):
                d1 = dst_v[pl.ds(i * 32, 16)]
                d2 = dst_v[pl.ds(i * 32 + 16, 16)]
                m1 = (d1 >= lo) & (d1 < lo + CR)
                m2 = (d2 >= lo) & (d2 < lo + CR)
                cs1 = plsc.cumsum(jnp.where(m1, jnp.int32(1), jnp.int32(0)))
                cs2 = plsc.cumsum(jnp.where(m2, jnp.int32(1), jnp.int32(0)))
                c1 = cs1[15]
                at1 = pos + cs1 - 1
                at2 = pos + c1 + cs2 - 1
                plsc.store_scatter(cdst2, [at1 >> 7, at1 & (G - 1)], d1 - lo,
                                   mask=m1)
                plsc.store_scatter(cdst2, [at2 >> 7, at2 & (G - 1)], d2 - lo,
                                   mask=m2)
                return pos + c1 + cs2[15]
            pos = lax.fori_loop(0, NVREG // 2, cmp_body, jnp.int32(0))
            nblk = (pos + (G - 1)) // G

            def tail(kk, _):
                at = pos + kk * 16 + iota
                mm = at < nblk * G
                plsc.store_scatter(cdst2, [at >> 7, at & (G - 1)], dsplat,
                                   mask=mm)
                return 0
            lax.fori_loop(0, G // 16, tail, 0)

            for i in range(STRIPE // ZR):
                pltpu.make_async_copy(
                    zbuf, acc.at[pl.ds(s * STRIPE + i * ZR, ZR)], zsem).wait()

            @pl.when(s == 0)
            def _():
                pltpu.make_async_copy(
                    zbuf.at[pl.ds(0, NS)], acc.at[pl.ds(CR, NS)], zsem).wait()
            plsc.subcore_barrier()

            def gs(b, _):
                pltpu.sync_copy(ones, acc.at[cdst2.at[b]], add=True)
                return 0
            lax.fori_loop(0, nblk, gs, 0)
            plsc.subcore_barrier()
            pltpu.sync_copy(acc.at[pl.ds(s * STRIPE, STRIPE)],
                            out.at[pl.ds(lo + s * STRIPE, STRIPE)])
            plsc.subcore_barrier()
    return k


def _enc(x, w, b):
    n, f = x.shape

    def body(x_ref, w_ref, b_ref, o_ref):
        o_ref[...] = jnp.maximum(
            jnp.dot(x_ref[...], w_ref[...], preferred_element_type=jnp.float32)
            + b_ref[...], 0.0)

    return pl.pallas_call(
        body,
        grid=(n // BR,),
        in_specs=[pl.BlockSpec((BR, f), lambda i: (i, 0)),
                  pl.BlockSpec((f, H), lambda i: (0, 0)),
                  pl.BlockSpec((1, H), lambda i: (0, 0))],
        out_specs=pl.BlockSpec((BR, H), lambda i: (i, 0)),
        out_shape=jax.ShapeDtypeStruct((n, H), jnp.float32),
    )(x, w, b)


def _combine_base(a0, a1, a2, a3, c0, c1, xb, wc, bs, wr, w1, b1, w2, b2):
    n = xb.shape[0]

    def body(a0r, a1r, a2r, a3r, c0r, c1r, xr, wcr, bsr, wrr, w1r, b1r, w2r,
             b2r, o_ref):
        r0 = jnp.maximum(c0r[...][:, :1], 1.0)
        r1 = jnp.maximum(c1r[...][:, :1], 1.0)
        acat = jnp.concatenate(
            [a0r[...] / r0, a1r[...] / r1, a2r[...], a3r[...]], axis=1)
        ob = (jnp.dot(acat, wcr[...], preferred_element_type=jnp.float32)
              + bsr[...]
              + jnp.dot(xr[...], wrr[...], preferred_element_type=jnp.float32))
        h = jnp.maximum(
            jnp.dot(ob, w1r[...], preferred_element_type=jnp.float32)
            + b1r[...], 0.0)
        o_ref[...] = (jnp.dot(h, w2r[...], preferred_element_type=jnp.float32)
                      + b2r[...] + xr[...])

    blk = lambda r, cdim: pl.BlockSpec((r, cdim), lambda i: (i, 0))
    full = lambda r, cdim: pl.BlockSpec((r, cdim), lambda i: (0, 0))
    return pl.pallas_call(
        body,
        grid=(n // BR,),
        in_specs=[blk(BR, H), blk(BR, H), blk(BR, H), blk(BR, H),
                  blk(BR, 16), blk(BR, 16), blk(BR, H),
                  full(4 * H, H), full(1, H), full(H, H),
                  full(H, H), full(1, H), full(H, H), full(1, H)],
        out_specs=blk(BR, H),
        out_shape=jax.ShapeDtypeStruct((n, H), jnp.float32),
    )(a0, a1, a2, a3, c0, c1, xb, wc, bs, wr, w1, b1, w2, b2)


def _combine_joint(a0, a1, xj, wc, bs, wr):
    n = xj.shape[0]

    def body(a0r, a1r, xr, wcr, bsr, wrr, o_ref):
        acat = jnp.concatenate([a0r[...], a1r[...]], axis=1)
        oj = (jnp.dot(acat, wcr[...], preferred_element_type=jnp.float32)
              + bsr[...]
              + jnp.dot(xr[...], wrr[...], preferred_element_type=jnp.float32))
        o_ref[...] = jnp.maximum(oj, 0.0) + xr[...]

    blk = lambda r, cdim: pl.BlockSpec((r, cdim), lambda i: (i, 0))
    full = lambda r, cdim: pl.BlockSpec((r, cdim), lambda i: (0, 0))
    return pl.pallas_call(
        body,
        grid=(n // BR,),
        in_specs=[blk(BR, H), blk(BR, H), blk(BR, H),
                  full(2 * H, H), full(1, H), full(H, H)],
        out_specs=blk(BR, H),
        out_shape=jax.ShapeDtypeStruct((n, H), jnp.float32),
    )(a0, a1, xj, wc, bs, wr)


def _decode(xb, w, b):
    n = xb.shape[0]

    def body(x_ref, w_ref, b_ref, o_ref):
        o_ref[...] = (jnp.dot(x_ref[...], w_ref[...],
                              preferred_element_type=jnp.float32) + b_ref[...])

    return pl.pallas_call(
        body,
        grid=(n // BR,),
        in_specs=[pl.BlockSpec((BR, H), lambda i: (i, 0)),
                  pl.BlockSpec((H, 8), lambda i: (0, 0)),
                  pl.BlockSpec((1, 8), lambda i: (0, 0))],
        out_specs=pl.BlockSpec((BR, 8), lambda i: (i, 0)),
        out_shape=jax.ShapeDtypeStruct((n, 8), jnp.float32),
    )(xb, w, b)


def kernel(x_base, x_joint, ei_gt, ei_gs, ei_gr, ei_jb, ei_bj, ei_jj,
           enc_W_base, enc_b_base, enc_W_joint, enc_b_joint,
           conv_W_rel, conv_b_rel, conv_W_root,
           bt_W1, bt_b1, bt_W2, bt_b2, dec_W, dec_b):
    nb = x_base.shape[0]
    nj = x_joint.shape[0]

    xb = _enc(jnp.pad(x_base, ((0, NB_PAD - nb), (0, 8 - x_base.shape[1]))),
              jnp.pad(enc_W_base, ((0, 8 - enc_W_base.shape[0]), (0, 0))),
              enc_b_base[None])
    xj = _enc(jnp.pad(x_joint, ((0, NJ_PAD - nj), (0, 8 - x_joint.shape[1]))),
              jnp.pad(enc_W_joint, ((0, 8 - enc_W_joint.shape[0]), (0, 0))),
              enc_b_joint[None])

    def prep(ei, sentinel):
        src = jnp.pad(ei[0], (0, E_PAD - E))
        dst = jnp.pad(ei[1], (0, E_PAD - E), constant_values=sentinel)
        return src, dst

    sgt = prep(ei_gt, NB_PAD)
    sgs = prep(ei_gs, NB_PAD)
    sgr = prep(ei_gr, NB_PAD)
    sjb = prep(ei_jb, NB_PAD)
    sbj = prep(ei_bj, NJ_PAD)
    sjj = prep(ei_jj, NJ_PAD)

    cgt = _seg_count(NB_PAD)(sgt[1])
    cgs = _seg_count(NB_PAD)(sgs[1])

    for l in range(2):
        a_gt = _seg_sum(NB_PAD, NB_PAD)(xb, *sgt)
        a_gs = _seg_sum(NB_PAD, NB_PAD)(xb, *sgs)
        a_gr = _seg_sum(NB_PAD, NB_PAD)(xb, *sgr)
        a_jb = _seg_sum(NJ_PAD, NB_PAD)(xj, *sjb)
        a_bj = _seg_sum(NB_PAD, NJ_PAD)(xb, *sbj)
        a_jj = _seg_sum(NJ_PAD, NJ_PAD)(xj, *sjj)
        wcb = jnp.concatenate([conv_W_rel[l, 0], conv_W_rel[l, 1],
                               conv_W_rel[l, 2], conv_W_rel[l, 3]], 0)
        wcj = jnp.concatenate([conv_W_rel[l, 4], conv_W_rel[l, 5]], 0)
        bsb = (conv_b_rel[l, 0] + conv_b_rel[l, 1] + conv_b_rel[l, 2]
               + conv_b_rel[l, 3])[None]
        bsj = (conv_b_rel[l, 4] + conv_b_rel[l, 5])[None]
        wrb = (conv_W_root[l, 0] + conv_W_root[l, 1] + conv_W_root[l, 2]
               + conv_W_root[l, 3])
        wrj = conv_W_root[l, 4] + conv_W_root[l, 5]
        xb = _combine_base(a_gt, a_gs, a_gr, a_jb, cgt, cgs, xb, wcb, bsb,
                           wrb, bt_W1, bt_b1[None], bt_W2, bt_b2[None])
        xj = _combine_joint(a_bj, a_jj, xj, wcj, bsj, wrj)

    o = _decode(xb, jnp.pad(dec_W, ((0, 0), (0, 2))),
                jnp.pad(dec_b, (0, 2))[None])
    return o[:nb, :6].reshape(-1, 4, 6)
